# initial kernel scaffold (unmeasured)
import jax
import jax.numpy as jnp
from jax import lax
from jax.experimental import pallas as pl
from jax.experimental.pallas import tpu as pltpu

M = 4096
N = 4096
CH = 512
EPS = 1e-6


def kernel(partial, gamma):
    gamma2 = gamma.reshape(1, N)

    def body(partial_ref, gamma_ref, out_ref,
             recv_ref, a_ref, b_ref, copy_sems, send_sem, recv_sem):
        my_x = lax.axis_index("x")
        my_y = lax.axis_index("y")
        my_z = lax.axis_index("z")
        peer_x = 1 - my_x

        barrier_sem = pltpu.get_barrier_semaphore()
        pl.semaphore_signal(
            barrier_sem, inc=1,
            device_id=(peer_x, my_y, my_z),
            device_id_type=pl.DeviceIdType.MESH,
        )
        pl.semaphore_wait(barrier_sem, 1)

        rdma = pltpu.make_async_remote_copy(
            src_ref=partial_ref.at[0, pl.ds(peer_x * M, M), :],
            dst_ref=recv_ref,
            send_sem=send_sem,
            recv_sem=recv_sem,
            device_id=(peer_x, my_y, my_z),
            device_id_type=pl.DeviceIdType.MESH,
        )
        rdma.start()
        rdma.wait()

        for c in range(M // CH):
            r0 = c * CH
            ca = pltpu.make_async_copy(
                partial_ref.at[0, pl.ds(my_x * M + r0, CH), :],
                a_ref, copy_sems.at[0])
            cb = pltpu.make_async_copy(
                recv_ref.at[pl.ds(r0, CH), :], b_ref, copy_sems.at[1])
            ca.start()
            cb.start()
            ca.wait()
            cb.wait()
            y = a_ref[...] + b_ref[...]
            ms = jnp.mean(y * y, axis=1, keepdims=True)
            out_ref[pl.ds(r0, CH), :] = y * lax.rsqrt(ms + EPS) * gamma_ref[...]

    return pl.pallas_call(
        body,
        out_shape=jax.ShapeDtypeStruct((M, N), jnp.float32),
        in_specs=[
            pl.BlockSpec(memory_space=pl.ANY),
            pl.BlockSpec(memory_space=pltpu.VMEM),
        ],
        out_specs=pl.BlockSpec(memory_space=pltpu.VMEM),
        scratch_shapes=[
            pltpu.MemorySpace.HBM((M, N), jnp.float32),
            pltpu.VMEM((CH, N), jnp.float32),
            pltpu.VMEM((CH, N), jnp.float32),
            pltpu.SemaphoreType.DMA((2,)),
            pltpu.SemaphoreType.DMA,
            pltpu.SemaphoreType.DMA,
        ],
        compiler_params=pltpu.CompilerParams(collective_id=0),
    )(partial, gamma2)


# baseline (device time: 872284 ns/iter reference)
import jax
import jax.numpy as jnp
from jax import lax
from jax.experimental import pallas as pl
from jax.experimental.pallas import tpu as pltpu

M = 4096
N = 4096
CH = 256
EPS = 1e-6


def kernel(partial, gamma):
    gamma2 = gamma.reshape(1, N)

    def body(partial_ref, gamma_ref, out_ref, recv_ref,
             a_ref, b_ref, o_ref, copy_sems, send_sem, recv_sem):
        my_x = lax.axis_index("x")
        my_y = lax.axis_index("y")
        my_z = lax.axis_index("z")
        peer_x = 1 - my_x

        barrier_sem = pltpu.get_barrier_semaphore()
        pl.semaphore_signal(
            barrier_sem, inc=1,
            device_id=(peer_x, my_y, my_z),
            device_id_type=pl.DeviceIdType.MESH,
        )
        pl.semaphore_wait(barrier_sem, 1)

        rdma = pltpu.make_async_remote_copy(
            src_ref=partial_ref.at[0, pl.ds(peer_x * M, M), :],
            dst_ref=recv_ref,
            send_sem=send_sem,
            recv_sem=recv_sem,
            device_id=(peer_x, my_y, my_z),
            device_id_type=pl.DeviceIdType.MESH,
        )
        rdma.start()
        rdma.wait()

        for c in range(M // CH):
            r0 = c * CH
            ca = pltpu.make_async_copy(
                partial_ref.at[0, pl.ds(my_x * M + r0, CH), :],
                a_ref, copy_sems.at[0])
            cb = pltpu.make_async_copy(
                recv_ref.at[pl.ds(r0, CH), :], b_ref, copy_sems.at[1])
            ca.start()
            cb.start()
            ca.wait()
            cb.wait()
            y = a_ref[...] + b_ref[...]
            ms = jnp.mean(y * y, axis=1, keepdims=True)
            o_ref[...] = y * lax.rsqrt(ms + EPS) * gamma_ref[...]
            co = pltpu.make_async_copy(
                o_ref, out_ref.at[pl.ds(r0, CH), :], copy_sems.at[2])
            co.start()
            co.wait()

    out, _recv = pl.pallas_call(
        body,
        out_shape=[
            jax.ShapeDtypeStruct((M, N), jnp.float32),
            jax.ShapeDtypeStruct((M, N), jnp.float32),
        ],
        in_specs=[
            pl.BlockSpec(memory_space=pl.ANY),
            pl.BlockSpec(memory_space=pltpu.VMEM),
        ],
        out_specs=[
            pl.BlockSpec(memory_space=pl.ANY),
            pl.BlockSpec(memory_space=pl.ANY),
        ],
        scratch_shapes=[
            pltpu.VMEM((CH, N), jnp.float32),
            pltpu.VMEM((CH, N), jnp.float32),
            pltpu.VMEM((CH, N), jnp.float32),
            pltpu.SemaphoreType.DMA((3,)),
            pltpu.SemaphoreType.DMA,
            pltpu.SemaphoreType.DMA,
        ],
        compiler_params=pltpu.CompilerParams(collective_id=0),
    )(partial, gamma2)
    return out


# device time: 234876 ns/iter; 3.7138x vs baseline; 3.7138x over previous
import jax
import jax.numpy as jnp
from jax import lax
from jax.experimental import pallas as pl
from jax.experimental.pallas import tpu as pltpu

M = 4096
N = 4096
HALF = 2048
CH = 256
NC = HALF // CH
EPS = 1e-6


def kernel(partial, gamma):
    gamma2 = gamma.reshape(1, N)

    def body(partial_ref, gamma_ref, out_ref,
             xsend_ref, xrecv_ref, af_ref, ob_ref,
             ld_sems, od_sems, xs_sems, xr_sems, zs_sems, zr_sems):
        my_x = lax.axis_index("x")
        my_y = lax.axis_index("y")
        my_z = lax.axis_index("z")
        xpeer = (1 - my_x, my_y, my_z)
        zpeer = (my_x, my_y, 1 - my_z)

        barrier_sem = pltpu.get_barrier_semaphore()
        for nbr in (xpeer, zpeer):
            pl.semaphore_signal(barrier_sem, inc=1, device_id=nbr,
                                device_id_type=pl.DeviceIdType.MESH)
        pl.semaphore_wait(barrier_sem, 2)

        send_base = (1 - my_x) * M + my_z * HALF
        mine_base = my_x * M + my_z * HALF
        out_mine = my_z * HALF

        def x_rdma(c):
            return pltpu.make_async_remote_copy(
                src_ref=xsend_ref.at[c], dst_ref=xrecv_ref.at[c],
                send_sem=xs_sems.at[c], recv_sem=xr_sems.at[c],
                device_id=xpeer, device_id_type=pl.DeviceIdType.MESH)

        def z_rdma(c, slot):
            return pltpu.make_async_remote_copy(
                src_ref=ob_ref.at[slot],
                dst_ref=out_ref.at[pl.ds(out_mine + c * CH, CH), :],
                send_sem=zs_sems.at[c], recv_sem=zr_sems.at[c],
                device_id=zpeer, device_id_type=pl.DeviceIdType.MESH)

        def out_dma(c, slot):
            return pltpu.make_async_copy(
                ob_ref.at[slot],
                out_ref.at[pl.ds(out_mine + c * CH, CH), :],
                od_sems.at[slot])

        for c in range(NC):
            slot = c % 2
            cp = pltpu.make_async_copy(
                partial_ref.at[0, pl.ds(send_base + c * CH, CH), :],
                af_ref.at[slot], ld_sems.at[slot])
            cp.start()
            cp.wait()
            xsend_ref[c] = af_ref[slot].astype(jnp.bfloat16)
            x_rdma(c).start()

        for c in range(NC):
            slot = c % 2
            cp = pltpu.make_async_copy(
                partial_ref.at[0, pl.ds(mine_base + c * CH, CH), :],
                af_ref.at[slot], ld_sems.at[slot])
            cp.start()
            x_rdma(c).wait_recv()
            cp.wait()
            y = af_ref[slot][...] + xrecv_ref[c].astype(jnp.float32)
            ms = jnp.mean(y * y, axis=1, keepdims=True)
            o = y * lax.rsqrt(ms + EPS) * gamma_ref[...]
            if c >= 2:
                out_dma(c - 2, slot).wait()
                z_rdma(c - 2, slot).wait_send()
            ob_ref[slot] = o.astype(jnp.bfloat16)
            out_dma(c, slot).start()
            z_rdma(c, slot).start()

        for c in range(NC - 2, NC):
            out_dma(c, c % 2).wait()
            z_rdma(c, c % 2).wait_send()
        for c in range(NC):
            x_rdma(c).wait_send()
            z_rdma(c, 0).wait_recv()

    out = pl.pallas_call(
        body,
        out_shape=jax.ShapeDtypeStruct((M, N), jnp.bfloat16),
        in_specs=[
            pl.BlockSpec(memory_space=pl.ANY),
            pl.BlockSpec(memory_space=pltpu.VMEM),
        ],
        out_specs=pl.BlockSpec(memory_space=pl.ANY),
        scratch_shapes=[
            pltpu.VMEM((NC, CH, N), jnp.bfloat16),
            pltpu.VMEM((NC, CH, N), jnp.bfloat16),
            pltpu.VMEM((2, CH, N), jnp.float32),
            pltpu.VMEM((2, CH, N), jnp.bfloat16),
            pltpu.SemaphoreType.DMA((2,)),
            pltpu.SemaphoreType.DMA((2,)),
            pltpu.SemaphoreType.DMA((NC,)),
            pltpu.SemaphoreType.DMA((NC,)),
            pltpu.SemaphoreType.DMA((NC,)),
            pltpu.SemaphoreType.DMA((NC,)),
        ],
        compiler_params=pltpu.CompilerParams(
            collective_id=0,
            vmem_limit_bytes=60 * 1024 * 1024,
        ),
    )(partial, gamma2)
    return out


# device time: 181399 ns/iter; 4.8086x vs baseline; 1.2948x over previous
import jax
import jax.numpy as jnp
from jax import lax
from jax.experimental import pallas as pl
from jax.experimental.pallas import tpu as pltpu

M = 4096
N = 4096
Q = 1024
CH = 128
NC = Q // CH
EPS = 1e-6


def kernel(partial, gamma):
    gamma2 = gamma.reshape(1, N)

    def body(partial_ref, gamma_ref, out_ref,
             xsend_ref, xrecv_ref, af_ref, ob_ref,
             ld_sems, od_sems, xs_sems, xr_sems,
             ys_sems, yr_sems, zs_sems, zr_sems,
             fys_sems, fyr_sems, fzs_sems, fzr_sems):
        my_x = lax.axis_index("x")
        my_y = lax.axis_index("y")
        my_z = lax.axis_index("z")
        xpeer = (1 - my_x, my_y, my_z)
        ypeer = (my_x, 1 - my_y, my_z)
        zpeer = (my_x, my_y, 1 - my_z)

        barrier_sem = pltpu.get_barrier_semaphore()
        for nbr in (xpeer, ypeer, zpeer):
            pl.semaphore_signal(barrier_sem, inc=1, device_id=nbr,
                                device_id_type=pl.DeviceIdType.MESH)
        pl.semaphore_wait(barrier_sem, 3)

        mybase = my_y * 2048 + my_z * Q
        ybase = (1 - my_y) * 2048 + my_z * Q
        zbase = my_y * 2048 + (1 - my_z) * Q

        def x_rdma(c):
            return pltpu.make_async_remote_copy(
                src_ref=xsend_ref.at[c], dst_ref=xrecv_ref.at[c],
                send_sem=xs_sems.at[c], recv_sem=xr_sems.at[c],
                device_id=xpeer, device_id_type=pl.DeviceIdType.MESH)

        def yprim(c, slot):
            return pltpu.make_async_remote_copy(
                src_ref=ob_ref.at[slot],
                dst_ref=out_ref.at[pl.ds(mybase + c * CH, CH), :],
                send_sem=ys_sems.at[c], recv_sem=yr_sems.at[c],
                device_id=ypeer, device_id_type=pl.DeviceIdType.MESH)

        def zprim(c, slot):
            return pltpu.make_async_remote_copy(
                src_ref=ob_ref.at[slot],
                dst_ref=out_ref.at[pl.ds(mybase + c * CH, CH), :],
                send_sem=zs_sems.at[c], recv_sem=zr_sems.at[c],
                device_id=zpeer, device_id_type=pl.DeviceIdType.MESH)

        def yfwd(c):
            rows = pl.ds(zbase + c * CH, CH)
            return pltpu.make_async_remote_copy(
                src_ref=out_ref.at[rows, :], dst_ref=out_ref.at[rows, :],
                send_sem=fys_sems.at[c // 2], recv_sem=fyr_sems.at[c // 2],
                device_id=ypeer, device_id_type=pl.DeviceIdType.MESH)

        def zfwd(c):
            rows = pl.ds(ybase + c * CH, CH)
            return pltpu.make_async_remote_copy(
                src_ref=out_ref.at[rows, :], dst_ref=out_ref.at[rows, :],
                send_sem=fzs_sems.at[c // 2], recv_sem=fzr_sems.at[c // 2],
                device_id=zpeer, device_id_type=pl.DeviceIdType.MESH)

        def out_dma(c, slot):
            return pltpu.make_async_copy(
                ob_ref.at[slot],
                out_ref.at[pl.ds(mybase + c * CH, CH), :],
                od_sems.at[slot])

        for c in range(NC):
            slot = c % 2
            cp = pltpu.make_async_copy(
                partial_ref.at[0, pl.ds((1 - my_x) * M + mybase + c * CH, CH), :],
                af_ref.at[slot], ld_sems.at[slot])
            cp.start()
            cp.wait()
            xsend_ref[c] = af_ref[slot].astype(jnp.bfloat16)
            x_rdma(c).start()

        for c in range(NC):
            slot = c % 2
            cp = pltpu.make_async_copy(
                partial_ref.at[0, pl.ds(my_x * M + mybase + c * CH, CH), :],
                af_ref.at[slot], ld_sems.at[slot])
            cp.start()
            x_rdma(c).wait_recv()
            cp.wait()
            y = af_ref[slot][...] + xrecv_ref[c].astype(jnp.float32)
            ms = jnp.mean(y * y, axis=1, keepdims=True)
            o = y * lax.rsqrt(ms + EPS) * gamma_ref[...]
            if c >= 2:
                out_dma(c - 2, slot).wait()
                yprim(c - 2, slot).wait_send()
                zprim(c - 2, slot).wait_send()
            ob_ref[slot] = o.astype(jnp.bfloat16)
            out_dma(c, slot).start()
            yprim(c, slot).start()
            zprim(c, slot).start()
            if c >= 1:
                p = c - 1
                if p % 2 == 0:
                    zprim(p, 0).wait_recv()
                    yfwd(p).start()
                else:
                    yprim(p, 0).wait_recv()
                    zfwd(p).start()

        p = NC - 1
        yprim(p, 0).wait_recv()
        zfwd(p).start()
        for c in range(0, NC, 2):
            yprim(c, 0).wait_recv()
            zprim(c + 1, 0).wait_recv()
        for c in range(NC - 2, NC):
            out_dma(c, c % 2).wait()
            yprim(c, c % 2).wait_send()
            zprim(c, c % 2).wait_send()
        for c in range(NC):
            x_rdma(c).wait_send()
        for c in range(0, NC, 2):
            yfwd(c).wait_send()
            zfwd(c + 1).wait_send()
            yfwd(c).wait_recv()
            zfwd(c + 1).wait_recv()

    out = pl.pallas_call(
        body,
        out_shape=jax.ShapeDtypeStruct((M, N), jnp.bfloat16),
        in_specs=[
            pl.BlockSpec(memory_space=pl.ANY),
            pl.BlockSpec(memory_space=pltpu.VMEM),
        ],
        out_specs=pl.BlockSpec(memory_space=pl.ANY),
        scratch_shapes=[
            pltpu.VMEM((NC, CH, N), jnp.bfloat16),
            pltpu.VMEM((NC, CH, N), jnp.bfloat16),
            pltpu.VMEM((2, CH, N), jnp.float32),
            pltpu.VMEM((2, CH, N), jnp.bfloat16),
            pltpu.SemaphoreType.DMA((2,)),
            pltpu.SemaphoreType.DMA((2,)),
            pltpu.SemaphoreType.DMA((NC,)),
            pltpu.SemaphoreType.DMA((NC,)),
            pltpu.SemaphoreType.DMA((NC,)),
            pltpu.SemaphoreType.DMA((NC,)),
            pltpu.SemaphoreType.DMA((NC,)),
            pltpu.SemaphoreType.DMA((NC,)),
            pltpu.SemaphoreType.DMA((NC // 2,)),
            pltpu.SemaphoreType.DMA((NC // 2,)),
            pltpu.SemaphoreType.DMA((NC // 2,)),
            pltpu.SemaphoreType.DMA((NC // 2,)),
        ],
        compiler_params=pltpu.CompilerParams(
            collective_id=0,
            vmem_limit_bytes=48 * 1024 * 1024,
        ),
    )(partial, gamma2)
    return out
